# baseline (speedup 1.0000x reference)
import functools

import jax
import jax.numpy as jnp
from jax import lax
from jax.experimental import pallas as pl
from jax.experimental.pallas import tpu as pltpu
from jax.experimental.pallas import tpu_sc as plsc

_VOCAB = 1000000
_EMBED = 64
_MODES = 32


_NC, _NS = 2, 16
_NW = _NC * _NS
_CH = 128


def _make_gather(n_tok):
    per_w = n_tok // _NW
    n_ch = per_w // _CH
    mesh = plsc.VectorSubcoreMesh(core_axis_name="c", subcore_axis_name="s")

    @functools.partial(
        pl.kernel,
        mesh=mesh,
        compiler_params=pltpu.CompilerParams(
            use_tc_tiling_on_sc=False, needs_layout_passes=False
        ),
        out_type=[
            jax.ShapeDtypeStruct((n_tok, _MODES), jnp.float32),
            jax.ShapeDtypeStruct((n_tok, _MODES), jnp.float32),
        ],
        scratch_types=[
            pltpu.VMEM((_CH,), jnp.int32),
            pltpu.VMEM((_CH, _MODES), jnp.float32),
            pltpu.VMEM((_CH, _MODES), jnp.float32),
            pltpu.SemaphoreType.DMA,
            pltpu.SemaphoreType.DMA,
        ],
    )
    def gather_k(fr_hbm, fi_hbm, idx_hbm, or_hbm, oi_hbm,
                 idx_v, rr_v, ri_v, sem_r, sem_i):
        wid = lax.axis_index("s") * _NC + lax.axis_index("c")
        base = wid * per_w

        def chunk(i, carry):
            off = base + i * _CH
            pltpu.sync_copy(idx_hbm.at[pl.ds(off, _CH)], idx_v)
            cr = pltpu.async_copy(fr_hbm.at[idx_v], rr_v, sem_r)
            ci = pltpu.async_copy(fi_hbm.at[idx_v], ri_v, sem_i)
            cr.wait()
            pltpu.sync_copy(rr_v, or_hbm.at[pl.ds(off, _CH)])
            ci.wait()
            pltpu.sync_copy(ri_v, oi_hbm.at[pl.ds(off, _CH)])
            return carry

        lax.fori_loop(0, n_ch, chunk, 0)

    return gather_k


_RBLK = 6400


def _mix_body(rr_ref, ri_ref, m_ref, b_ref, out_ref):
    x = jnp.concatenate(
        [rr_ref[...].astype(jnp.bfloat16), ri_ref[...].astype(jnp.bfloat16)],
        axis=1,
    )
    acc = jnp.dot(x, m_ref[...], preferred_element_type=jnp.float32)
    out_ref[...] = acc + b_ref[...]


def _mix(rows_r, rows_i, a_mat, bias, n_tok):
    return pl.pallas_call(
        _mix_body,
        grid=(n_tok // _RBLK,),
        in_specs=[
            pl.BlockSpec((_RBLK, _MODES), lambda i: (i, 0)),
            pl.BlockSpec((_RBLK, _MODES), lambda i: (i, 0)),
            pl.BlockSpec((2 * _MODES, _EMBED), lambda i: (0, 0)),
            pl.BlockSpec((1, _EMBED), lambda i: (0, 0)),
        ],
        out_specs=pl.BlockSpec((_RBLK, _EMBED), lambda i: (i, 0)),
        out_shape=jax.ShapeDtypeStruct((n_tok, _EMBED), jnp.float32),
    )(rows_r, rows_i, a_mat, bias)


def kernel(tokens, freq_real, freq_imag, mode_weights, phase, W, b):
    w = jax.nn.softplus(mode_weights)
    c = jnp.cos(phase)
    s = jnp.sin(phase)
    w1t = W[:, :_MODES].T
    w2t = W[:, _MODES:].T
    a_real = (w * c)[:, None] * w1t + (w * s)[:, None] * w2t
    a_imag = (w * c)[:, None] * w2t - (w * s)[:, None] * w1t
    a_mat = jnp.concatenate([a_real, a_imag], axis=0).astype(jnp.bfloat16)
    bias = b.reshape(1, _EMBED)

    bsz, tsz = tokens.shape
    n_tok = bsz * tsz
    idx = tokens.reshape(-1).astype(jnp.int32)
    rows_r, rows_i = _make_gather(n_tok)(freq_real, freq_imag, idx)
    out = _mix(rows_r, rows_i, a_mat, bias, n_tok)
    return out.reshape(bsz, tsz, _EMBED)


# speedup vs baseline: 1.0113x; 1.0113x over previous
import functools

import jax
import jax.numpy as jnp
from jax import lax
from jax.experimental import pallas as pl
from jax.experimental.pallas import tpu as pltpu
from jax.experimental.pallas import tpu_sc as plsc

_VOCAB = 1000000
_EMBED = 64
_MODES = 32


_BLK = 8000


def _transform_body(fr_ref, fi_ref, m_ref, b_ref, out_ref):
    x = jnp.concatenate(
        [fr_ref[...].astype(jnp.bfloat16), fi_ref[...].astype(jnp.bfloat16)],
        axis=1,
    )
    acc = jnp.dot(x, m_ref[...], preferred_element_type=jnp.float32)
    out_ref[...] = acc + b_ref[...]


def _build_table(freq_real, freq_imag, a_mat, bias):
    grid = (_VOCAB // _BLK,)
    return pl.pallas_call(
        _transform_body,
        grid=grid,
        in_specs=[
            pl.BlockSpec((_BLK, _MODES), lambda i: (i, 0)),
            pl.BlockSpec((_BLK, _MODES), lambda i: (i, 0)),
            pl.BlockSpec((2 * _MODES, _EMBED), lambda i: (0, 0)),
            pl.BlockSpec((1, _EMBED), lambda i: (0, 0)),
        ],
        out_specs=pl.BlockSpec((_BLK, _EMBED), lambda i: (i, 0)),
        out_shape=jax.ShapeDtypeStruct((_VOCAB, _EMBED), jnp.float32),
    )(freq_real, freq_imag, a_mat, bias)


_NC, _NS = 2, 16
_NW = _NC * _NS
_CH = 128


def _make_gather(n_tok):
    per_w = n_tok // _NW
    n_ch = per_w // _CH
    mesh = plsc.VectorSubcoreMesh(core_axis_name="c", subcore_axis_name="s")

    @functools.partial(
        pl.kernel,
        mesh=mesh,
        compiler_params=pltpu.CompilerParams(
            use_tc_tiling_on_sc=False, needs_layout_passes=False
        ),
        out_type=jax.ShapeDtypeStruct((n_tok, _EMBED), jnp.float32),
        scratch_types=[
            pltpu.VMEM((_CH,), jnp.int32),
            pltpu.VMEM((_CH, _EMBED), jnp.float32),
            pltpu.SemaphoreType.DMA,
        ],
    )
    def gather_k(table_hbm, idx_hbm, out_hbm, idx_v, rows_v, sem):
        wid = lax.axis_index("s") * _NC + lax.axis_index("c")
        base = wid * per_w

        def chunk(i, carry):
            off = base + i * _CH
            pltpu.sync_copy(idx_hbm.at[pl.ds(off, _CH)], idx_v)
            pltpu.async_copy(table_hbm.at[idx_v], rows_v, sem).wait()
            pltpu.sync_copy(rows_v, out_hbm.at[pl.ds(off, _CH)])
            return carry

        lax.fori_loop(0, n_ch, chunk, 0)

    return gather_k


def kernel(tokens, freq_real, freq_imag, mode_weights, phase, W, b):
    w = jax.nn.softplus(mode_weights)
    c = jnp.cos(phase)
    s = jnp.sin(phase)
    w1t = W[:, :_MODES].T
    w2t = W[:, _MODES:].T
    a_real = (w * c)[:, None] * w1t + (w * s)[:, None] * w2t
    a_imag = (w * c)[:, None] * w2t - (w * s)[:, None] * w1t
    a_mat = jnp.concatenate([a_real, a_imag], axis=0).astype(jnp.bfloat16)
    bias = b.reshape(1, _EMBED)

    table = _build_table(freq_real, freq_imag, a_mat, bias)

    bsz, tsz = tokens.shape
    idx = tokens.reshape(-1).astype(jnp.int32)
    out = _make_gather(bsz * tsz)(table, idx)
    return out.reshape(bsz, tsz, _EMBED)
